# Initial kernel scaffold; baseline (speedup 1.0000x reference)
import jax
import jax.numpy as jnp
from jax.experimental import pallas as pl


def kernel(x, tables, W, b):
    raise NotImplementedError("write your pallas kernel here")


# speedup vs baseline: 16.7998x; 16.7998x over previous
import functools

import jax
import jax.numpy as jnp
from jax import lax
from jax.experimental import pallas as pl
from jax.experimental.pallas import tpu as pltpu
from jax.experimental.pallas import tpu_sc as plsc

NC = 2
NS = 16
NW = NC * NS
CHUNK = 128


@functools.partial(jax.jit, static_argnames=("nchunks",))
def _sc_gather(idx3, table2d, *, nchunks):
    fv, d = table2d.shape
    n = NW * nchunks * CHUNK
    mesh = plsc.VectorSubcoreMesh(
        core_axis_name="c", subcore_axis_name="s", num_cores=NC, num_subcores=NS
    )

    @functools.partial(
        pl.kernel,
        mesh=mesh,
        compiler_params=pltpu.CompilerParams(use_tc_tiling_on_sc=False),
        out_type=jax.ShapeDtypeStruct((n, d), jnp.float32),
        scratch_types=[
            pltpu.VMEM((nchunks, CHUNK), jnp.int32),
            pltpu.VMEM((CHUNK, d), jnp.float32),
            pltpu.SemaphoreType.DMA,
        ],
    )
    def k(idx_hbm, tab_hbm, out_hbm, idx_v, buf, sem):
        wid = lax.axis_index("s") * NC + lax.axis_index("c")
        pltpu.sync_copy(idx_hbm.at[wid], idx_v)

        @pl.loop(0, nchunks)
        def _body(j):
            pltpu.async_copy(tab_hbm.at[idx_v.at[j]], buf, sem).wait()
            pltpu.sync_copy(buf, out_hbm.at[pl.ds((wid * nchunks + j) * CHUNK, CHUNK)])

    return k(idx3, table2d)


def _mm_bias(a, w, bias, bm):
    m, kdim = a.shape
    nout = w.shape[1]

    def body(a_ref, w_ref, b_ref, o_ref):
        o_ref[...] = (
            jnp.dot(a_ref[...], w_ref[...], preferred_element_type=jnp.float32)
            + b_ref[...]
        )

    return pl.pallas_call(
        body,
        grid=(m // bm,),
        in_specs=[
            pl.BlockSpec((bm, kdim), lambda i: (i, 0)),
            pl.BlockSpec((kdim, nout), lambda i: (0, 0)),
            pl.BlockSpec((1, nout), lambda i: (0, 0)),
        ],
        out_specs=pl.BlockSpec((bm, nout), lambda i: (i, 0)),
        out_shape=jax.ShapeDtypeStruct((m, nout), jnp.float32),
    )(a, w, bias.reshape(1, nout))


def kernel(x, tables, W, b):
    s, bsz, f = x.shape
    f2, v, d = tables.shape
    n = s * bsz * f
    assert n % (NW * CHUNK) == 0
    nchunks = n // (NW * CHUNK)
    flat_idx = (x.astype(jnp.int32) + jnp.arange(f, dtype=jnp.int32) * v).reshape(
        NW, nchunks, CHUNK
    )
    rows = _sc_gather(flat_idx, tables.reshape(f2 * v, d), nchunks=nchunks)
    a = rows.reshape(s * bsz, f * d)
    y = _mm_bias(a, W, b, bm=1024)
    return y.reshape(s, bsz, W.shape[1])
